# Initial kernel scaffold; baseline (speedup 1.0000x reference)
import jax
import jax.numpy as jnp
from jax.experimental import pallas as pl


def kernel(user_emb, item_emb, W_u, U_u, ub_u, ib_u, W_r, U_r, ub_r, ib_r, W_h, U_h, ub_h, ib_h, edge_index_series):
    raise NotImplementedError("write your pallas kernel here")


# speedup vs baseline: 9.7247x; 9.7247x over previous
import functools

import jax
import jax.numpy as jnp
from jax import lax
from jax.experimental import pallas as pl
from jax.experimental.pallas import tpu as pltpu
from jax.experimental.pallas import tpu_sc as plsc

NU = 5000
N = 10000
D = 128
T = 4
E = 320000
NC = 2
NT = 16
LN = 16

NP = 10240
RPT = NP // NT
EK = 128
CPT = 157
EP = NT * CPT * EK
EPT = CPT * EK
RC = 32
NRC = RPT // RC

BN = 512


def _mm(a, w):
    return lax.dot_general(a, w, (((1,), (1,)), ((), ())),
                           preferred_element_type=jnp.float32)


def _tc_prep_body(x0_ref, wu_ref, uu_ref, wr_ref, ur_ref, wh_ref, uh_ref,
                  bias_ref, xs_ref, oi_ref):
    i = pl.program_id(0)
    rows = lax.broadcasted_iota(jnp.int32, (BN, 1), 0) + i * BN
    umask = rows < NU
    vmask = rows < N
    b = bias_ref[...]
    bu = jnp.where(umask, b[0:1, :], b[1:2, :])
    br = jnp.where(umask, b[2:3, :], b[3:4, :])
    bh = jnp.where(umask, b[4:5, :], b[5:6, :])
    wuu = wu_ref[...] + uu_ref[...]
    wrr = wr_ref[...] + ur_ref[...]
    wh = wh_ref[...]
    uh = uh_ref[...]
    xt = x0_ref[...]
    xs_ref[0] = xt
    acc = xt
    for t in range(1, T):
        u = jax.nn.sigmoid(_mm(xt, wuu) + bu)
        r = jax.nn.sigmoid(_mm(xt, wrr) + br)
        hc = jnp.tanh(_mm(xt, wh) + _mm(r * xt, uh) + bh)
        xt = (1.0 - u) * xt + u * hc
        xt = jnp.where(vmask, xt, 0.0)
        xs_ref[t] = xt
        acc = acc + xt
    oi_ref[...] = acc * (1.0 / 12.0)


def _tc_prep(x0, wu, uu, wr, ur, wh, uh, bias):
    full = pl.BlockSpec((D, D), lambda i: (0, 0))
    return pl.pallas_call(
        _tc_prep_body,
        grid=(NP // BN,),
        in_specs=[pl.BlockSpec((BN, D), lambda i: (i, 0)),
                  full, full, full, full, full, full,
                  pl.BlockSpec((8, D), lambda i: (0, 0))],
        out_specs=[pl.BlockSpec((T, BN, D), lambda i: (0, i, 0)),
                   pl.BlockSpec((BN, D), lambda i: (i, 0))],
        out_shape=[jax.ShapeDtypeStruct((T, NP, D), jnp.float32),
                   jax.ShapeDtypeStruct((NP, D), jnp.float32)],
    )(x0, wu, uu, wr, ur, wh, uh, bias)


def _sc_main_body(xs, srcs, dsts, out, xp, y1p,
                  acc_sh, deg_sh, isrc, idst, rows, disb, onesb,
                  arow, brow, sem):
    c = lax.axis_index("c")
    s = lax.axis_index("s")
    r0 = s * RPT
    e0 = s * EPT

    half = jnp.full((LN,), 0.5, jnp.float32)
    three_half = jnp.full((LN,), 1.5, jnp.float32)
    magic = jnp.full((LN,), 0x5F3759DF, jnp.int32)
    zero16 = jnp.zeros((LN,), jnp.float32)
    one16 = jnp.ones((LN,), jnp.float32)

    def zero_brow():
        def f_zb(i, _):
            for v in range(D // LN):
                brow[i, pl.ds(v * LN, LN)] = zero16
            return _
        lax.fori_loop(0, RC, f_zb, None)

    def f_ones(i, _):
        onesb[i] = one16
        return _
    lax.fori_loop(0, EK, f_ones, None)
    zero_brow()

    def f_zout(k, _):
        pltpu.sync_copy(brow, out.at[c, pl.ds(r0 + k * RC, RC)])
        return _
    lax.fori_loop(0, NRC, f_zout, None)

    def per_t(tl, _):
        t = c * 2 + tl
        zero_brow()

        def f_zacc(k, _2):
            pltpu.sync_copy(brow, acc_sh.at[pl.ds(r0 + k * RC, RC)])
            return _2
        lax.fori_loop(0, NRC, f_zacc, None)

        def f_zdeg(i, _2):
            disb[i] = zero16
            return _2
        lax.fori_loop(0, RPT, f_zdeg, None)
        pltpu.sync_copy(disb, deg_sh.at[pl.ds(r0, RPT)])
        plsc.subcore_barrier()

        def f_deg(i, _2):
            pltpu.sync_copy(dsts.at[t, pl.ds(e0 + i * EK, EK)], idst)
            pltpu.sync_copy(onesb, deg_sh.at[idst], add=True)
            return _2
        lax.fori_loop(0, CPT, f_deg, None)
        plsc.subcore_barrier()

        pltpu.sync_copy(deg_sh.at[pl.ds(r0, RPT)], disb)

        def f_dis(i, _2):
            d = disb[i]
            ii = magic - (lax.bitcast_convert_type(d, jnp.int32) >> 1)
            y = lax.bitcast_convert_type(ii, jnp.float32)
            for _n in range(3):
                y = y * (three_half - half * d * y * y)
            disb[i] = jnp.where(d > half, y, zero16)
            return _2
        lax.fori_loop(0, RPT, f_dis, None)

        def f_xp(k, _2):
            rb = k * RC
            pltpu.sync_copy(xs.at[t, pl.ds(r0 + rb, RC)], arow)

            def f_r(r, _3):
                dv = disb[rb + r]
                for v in range(D // LN):
                    sl = pl.ds(v * LN, LN)
                    arow[r, sl] = arow[r, sl] * dv
                return _3
            lax.fori_loop(0, RC, f_r, None)
            pltpu.sync_copy(arow, xp.at[c, pl.ds(r0 + rb, RC)])
            return _2
        lax.fori_loop(0, NRC, f_xp, None)
        plsc.subcore_barrier()

        def conv(src_ref):
            def f_e(i, _2):
                base = e0 + i * EK
                pltpu.sync_copy(srcs.at[t, pl.ds(base, EK)], isrc)
                pltpu.sync_copy(dsts.at[t, pl.ds(base, EK)], idst)
                pltpu.async_copy(src_ref.at[c].at[isrc], rows, sem).wait()
                pltpu.sync_copy(rows, acc_sh.at[idst], add=True)
                return _2
            lax.fori_loop(0, CPT, f_e, None)
            plsc.subcore_barrier()

        conv(xp)

        def f_y1(k, _2):
            rb = k * RC
            pltpu.sync_copy(acc_sh.at[pl.ds(r0 + rb, RC)], arow)

            def f_r(r, _3):
                dv = disb[rb + r]
                d2 = dv * dv
                for v in range(D // LN):
                    sl = pl.ds(v * LN, LN)
                    arow[r, sl] = arow[r, sl] * d2
                return _3
            lax.fori_loop(0, RC, f_r, None)
            pltpu.sync_copy(arow, y1p.at[c, pl.ds(r0 + rb, RC)])
            pltpu.sync_copy(brow, acc_sh.at[pl.ds(r0 + rb, RC)])
            return _2
        lax.fori_loop(0, NRC, f_y1, None)
        plsc.subcore_barrier()

        conv(y1p)

        ob = rows.at[pl.ds(0, RC)]

        def f_fin(k, _2):
            rb = k * RC
            pltpu.sync_copy(acc_sh.at[pl.ds(r0 + rb, RC)], arow)
            pltpu.sync_copy(y1p.at[c, pl.ds(r0 + rb, RC)], brow)
            pltpu.sync_copy(out.at[c, pl.ds(r0 + rb, RC)], ob)

            def f_r(r, _3):
                dv = disb[rb + r]
                pos = dv > zero16
                for v in range(D // LN):
                    sl = pl.ds(v * LN, LN)
                    y1 = jnp.where(pos, brow[r, sl] / dv, zero16)
                    rows[r, sl] = rows[r, sl] + (arow[r, sl] * dv + y1) * (1.0 / 12.0)
                return _3
            lax.fori_loop(0, RC, f_r, None)
            pltpu.sync_copy(ob, out.at[c, pl.ds(r0 + rb, RC)])
            return _2
        lax.fori_loop(0, NRC, f_fin, None)
        return _
    lax.fori_loop(0, T // NC, per_t, None)


@functools.partial(
    pl.kernel,
    out_type=[jax.ShapeDtypeStruct((NC, NP, D), jnp.float32),
              jax.ShapeDtypeStruct((NC, NP, D), jnp.float32),
              jax.ShapeDtypeStruct((NC, NP, D), jnp.float32)],
    mesh=plsc.VectorSubcoreMesh(core_axis_name="c", subcore_axis_name="s",
                                num_cores=NC, num_subcores=NT),
    scratch_types=[
        pltpu.VMEM_SHARED((NP, D), jnp.float32),
        pltpu.VMEM_SHARED((NP, LN), jnp.float32),
        pltpu.VMEM((EK,), jnp.int32),
        pltpu.VMEM((EK,), jnp.int32),
        pltpu.VMEM((EK, D), jnp.float32),
        pltpu.VMEM((RPT, LN), jnp.float32),
        pltpu.VMEM((EK, LN), jnp.float32),
        pltpu.VMEM((RC, D), jnp.float32),
        pltpu.VMEM((RC, D), jnp.float32),
        pltpu.SemaphoreType.DMA,
    ],
    compiler_params=pltpu.CompilerParams(use_tc_tiling_on_sc=False),
)
def _sc_main(xs, srcs, dsts, out, xp, y1p, acc_sh, deg_sh, isrc, idst, rows,
             disb, onesb, arow, brow, sem):
    _sc_main_body(xs, srcs, dsts, out, xp, y1p, acc_sh, deg_sh, isrc, idst,
                  rows, disb, onesb, arow, brow, sem)


def _tc_fin_body(oi_ref, o0_ref, o1_ref, f_ref):
    f_ref[...] = oi_ref[...] + o0_ref[0] + o1_ref[0]


def _tc_fin(oi, outp):
    bf = 1000
    return pl.pallas_call(
        _tc_fin_body,
        grid=(N // bf,),
        in_specs=[pl.BlockSpec((bf, D), lambda i: (i, 0)),
                  pl.BlockSpec((1, bf, D), lambda i: (0, i, 0)),
                  pl.BlockSpec((1, bf, D), lambda i: (1, i, 0))],
        out_specs=pl.BlockSpec((bf, D), lambda i: (i, 0)),
        out_shape=jax.ShapeDtypeStruct((N, D), jnp.float32),
    )(oi, outp, outp)


def kernel(user_emb, item_emb, W_u, U_u, ub_u, ib_u, W_r, U_r, ub_r, ib_r,
           W_h, U_h, ub_h, ib_h, edge_index_series):
    f32 = jnp.float32
    x0 = jnp.concatenate(
        [user_emb.astype(f32), item_emb.astype(f32),
         jnp.zeros((NP - N, D), f32)], axis=0)
    bias = jnp.concatenate(
        [ub_u.reshape(1, D), ib_u.reshape(1, D),
         ub_r.reshape(1, D), ib_r.reshape(1, D),
         ub_h.reshape(1, D), ib_h.reshape(1, D),
         jnp.zeros((2, D), f32)], axis=0).astype(f32)

    xs, oi = _tc_prep(x0, W_u.astype(f32), U_u.astype(f32), W_r.astype(f32),
                      U_r.astype(f32), W_h.astype(f32), U_h.astype(f32), bias)

    ei = edge_index_series.astype(jnp.int32)
    pad = jnp.full((T, EP - E), N, jnp.int32)
    srcs = jnp.concatenate([ei[:, 0, :], pad], axis=1)
    dsts = jnp.concatenate([ei[:, 1, :], pad], axis=1)

    outp, _xp, _y1p = _sc_main(xs, srcs, dsts)

    fin = _tc_fin(oi, outp)
    return (fin[:NU], fin[NU:N])


# speedup vs baseline: 11.2073x; 1.1525x over previous
import functools

import jax
import jax.numpy as jnp
from jax import lax
from jax.experimental import pallas as pl
from jax.experimental.pallas import tpu as pltpu
from jax.experimental.pallas import tpu_sc as plsc

NU = 5000
N = 10000
D = 128
T = 4
E = 320000
NC = 2
NT = 16
LN = 16

NP = 10240
RPT = NP // NT
EK = 64
CPT = 314
NB = 2
EP = NT * CPT * EK
EPT = CPT * EK
RC = 32
NRC = RPT // RC

BN = 512


def _mm(a, w):
    return lax.dot_general(a, w, (((1,), (1,)), ((), ())),
                           preferred_element_type=jnp.float32)


def _tc_prep_body(x0_ref, wu_ref, uu_ref, wr_ref, ur_ref, wh_ref, uh_ref,
                  bias_ref, xs_ref, oi_ref):
    i = pl.program_id(0)
    rows = lax.broadcasted_iota(jnp.int32, (BN, 1), 0) + i * BN
    umask = rows < NU
    vmask = rows < N
    b = bias_ref[...]
    bu = jnp.where(umask, b[0:1, :], b[1:2, :])
    br = jnp.where(umask, b[2:3, :], b[3:4, :])
    bh = jnp.where(umask, b[4:5, :], b[5:6, :])
    wuu = wu_ref[...] + uu_ref[...]
    wrr = wr_ref[...] + ur_ref[...]
    wh = wh_ref[...]
    uh = uh_ref[...]
    xt = x0_ref[...]
    xs_ref[0] = xt
    acc = xt
    for t in range(1, T):
        u = jax.nn.sigmoid(_mm(xt, wuu) + bu)
        r = jax.nn.sigmoid(_mm(xt, wrr) + br)
        hc = jnp.tanh(_mm(xt, wh) + _mm(r * xt, uh) + bh)
        xt = (1.0 - u) * xt + u * hc
        xt = jnp.where(vmask, xt, 0.0)
        xs_ref[t] = xt
        acc = acc + xt
    oi_ref[...] = acc * (1.0 / 12.0)


def _tc_prep(x0, wu, uu, wr, ur, wh, uh, bias):
    full = pl.BlockSpec((D, D), lambda i: (0, 0))
    return pl.pallas_call(
        _tc_prep_body,
        grid=(NP // BN,),
        in_specs=[pl.BlockSpec((BN, D), lambda i: (i, 0)),
                  full, full, full, full, full, full,
                  pl.BlockSpec((8, D), lambda i: (0, 0))],
        out_specs=[pl.BlockSpec((T, BN, D), lambda i: (0, i, 0)),
                   pl.BlockSpec((BN, D), lambda i: (i, 0))],
        out_shape=[jax.ShapeDtypeStruct((T, NP, D), jnp.float32),
                   jax.ShapeDtypeStruct((NP, D), jnp.float32)],
    )(x0, wu, uu, wr, ur, wh, uh, bias)


def _sc_main_body(xs, srcs, dsts, out, xp, y1p,
                  acc_sh, deg_sh, isrc, idst, rows, disb, onesb,
                  arow, brow, sem0, sem1, isem0, isem1):
    c = lax.axis_index("c")
    s = lax.axis_index("s")
    r0 = s * RPT
    e0 = s * EPT

    half = jnp.full((LN,), 0.5, jnp.float32)
    three_half = jnp.full((LN,), 1.5, jnp.float32)
    magic = jnp.full((LN,), 0x5F3759DF, jnp.int32)
    zero16 = jnp.zeros((LN,), jnp.float32)
    one16 = jnp.ones((LN,), jnp.float32)

    def zero_brow():
        def f_zb(i, _):
            for v in range(D // LN):
                brow[i, pl.ds(v * LN, LN)] = zero16
            return _
        lax.fori_loop(0, RC, f_zb, None)

    def f_ones(i, _):
        onesb[i] = one16
        return _
    lax.fori_loop(0, EK, f_ones, None)
    zero_brow()

    def f_zout(k, _):
        pltpu.sync_copy(brow, out.at[c, pl.ds(r0 + k * RC, RC)])
        return _
    lax.fori_loop(0, NRC, f_zout, None)

    def per_t(tl, _):
        t = c * 2 + tl
        zero_brow()

        def f_zacc(k, _2):
            pltpu.sync_copy(brow, acc_sh.at[pl.ds(r0 + k * RC, RC)])
            return _2
        lax.fori_loop(0, NRC, f_zacc, None)

        def f_zdeg(i, _2):
            disb[i] = zero16
            return _2
        lax.fori_loop(0, RPT, f_zdeg, None)
        pltpu.sync_copy(disb, deg_sh.at[pl.ds(r0, RPT)])
        plsc.subcore_barrier()

        isems = (isem0, isem1)

        def dslice(i):
            return dsts.at[t, pl.ds(e0 + i * EK, EK)]

        for b in range(NB):
            pltpu.async_copy(dslice(b), idst.at[b], isems[b])

        def f_deg(g, _2):
            for b in range(NB):
                i = NB * g + b
                pltpu.make_async_copy(dslice(i), idst.at[b], isems[b]).wait()
                pltpu.sync_copy(onesb, deg_sh.at[idst.at[b]], add=True)
                pltpu.async_copy(dslice(i + NB), idst.at[b], isems[b])
            return _2
        lax.fori_loop(0, CPT // NB - 1, f_deg, None)
        for b in range(NB):
            i = CPT - NB + b
            pltpu.make_async_copy(dslice(i), idst.at[b], isems[b]).wait()
            pltpu.sync_copy(onesb, deg_sh.at[idst.at[b]], add=True)
        plsc.subcore_barrier()

        pltpu.sync_copy(deg_sh.at[pl.ds(r0, RPT)], disb)

        def f_dis(i, _2):
            d = disb[i]
            ii = magic - (lax.bitcast_convert_type(d, jnp.int32) >> 1)
            y = lax.bitcast_convert_type(ii, jnp.float32)
            for _n in range(3):
                y = y * (three_half - half * d * y * y)
            disb[i] = jnp.where(d > half, y, zero16)
            return _2
        lax.fori_loop(0, RPT, f_dis, None)

        def f_xp(k, _2):
            rb = k * RC
            pltpu.sync_copy(xs.at[t, pl.ds(r0 + rb, RC)], arow)

            def f_r(r, _3):
                dv = disb[rb + r]
                for v in range(D // LN):
                    sl = pl.ds(v * LN, LN)
                    arow[r, sl] = arow[r, sl] * dv
                return _3
            lax.fori_loop(0, RC, f_r, None)
            pltpu.sync_copy(arow, xp.at[c, pl.ds(r0 + rb, RC)])
            return _2
        lax.fori_loop(0, NRC, f_xp, None)
        plsc.subcore_barrier()

        gsems = (sem0, sem1)

        def conv(src_ref):
            def idx_fetch(i, b):
                pltpu.sync_copy(srcs.at[t, pl.ds(e0 + i * EK, EK)],
                                isrc.at[b])
                pltpu.sync_copy(dsts.at[t, pl.ds(e0 + i * EK, EK)],
                                idst.at[b])

            def gdesc(b):
                return pltpu.make_async_copy(src_ref.at[c].at[isrc.at[b]],
                                             rows.at[b], gsems[b])

            for b in range(NB):
                idx_fetch(b, b)
                pltpu.async_copy(src_ref.at[c].at[isrc.at[b]], rows.at[b],
                                 gsems[b])

            def f_e(g, _2):
                for b in range(NB):
                    i = NB * g + b
                    gdesc(b).wait()
                    pltpu.sync_copy(rows.at[b], acc_sh.at[idst.at[b]],
                                    add=True)
                    idx_fetch(i + NB, b)
                    pltpu.async_copy(src_ref.at[c].at[isrc.at[b]],
                                     rows.at[b], gsems[b])
                return _2
            lax.fori_loop(0, CPT // NB - 1, f_e, None)
            for b in range(NB):
                gdesc(b).wait()
                pltpu.sync_copy(rows.at[b], acc_sh.at[idst.at[b]], add=True)
            plsc.subcore_barrier()

        conv(xp)

        def f_y1(k, _2):
            rb = k * RC
            pltpu.sync_copy(acc_sh.at[pl.ds(r0 + rb, RC)], arow)

            def f_r(r, _3):
                dv = disb[rb + r]
                d2 = dv * dv
                for v in range(D // LN):
                    sl = pl.ds(v * LN, LN)
                    arow[r, sl] = arow[r, sl] * d2
                return _3
            lax.fori_loop(0, RC, f_r, None)
            pltpu.sync_copy(arow, y1p.at[c, pl.ds(r0 + rb, RC)])
            pltpu.sync_copy(brow, acc_sh.at[pl.ds(r0 + rb, RC)])
            return _2
        lax.fori_loop(0, NRC, f_y1, None)
        plsc.subcore_barrier()

        conv(y1p)

        ob = rows.at[0, pl.ds(0, RC)]

        def f_fin(k, _2):
            rb = k * RC
            pltpu.sync_copy(acc_sh.at[pl.ds(r0 + rb, RC)], arow)
            pltpu.sync_copy(y1p.at[c, pl.ds(r0 + rb, RC)], brow)
            pltpu.sync_copy(out.at[c, pl.ds(r0 + rb, RC)], ob)

            def f_r(r, _3):
                dv = disb[rb + r]
                pos = dv > zero16
                for v in range(D // LN):
                    sl = pl.ds(v * LN, LN)
                    y1 = jnp.where(pos, brow[r, sl] / dv, zero16)
                    rows[0, r, sl] = rows[0, r, sl] + (arow[r, sl] * dv + y1) * (1.0 / 12.0)
                return _3
            lax.fori_loop(0, RC, f_r, None)
            pltpu.sync_copy(ob, out.at[c, pl.ds(r0 + rb, RC)])
            return _2
        lax.fori_loop(0, NRC, f_fin, None)
        return _
    lax.fori_loop(0, T // NC, per_t, None)


@functools.partial(
    pl.kernel,
    out_type=[jax.ShapeDtypeStruct((NC, NP, D), jnp.float32),
              jax.ShapeDtypeStruct((NC, NP, D), jnp.float32),
              jax.ShapeDtypeStruct((NC, NP, D), jnp.float32)],
    mesh=plsc.VectorSubcoreMesh(core_axis_name="c", subcore_axis_name="s",
                                num_cores=NC, num_subcores=NT),
    scratch_types=[
        pltpu.VMEM_SHARED((NP, D), jnp.float32),
        pltpu.VMEM_SHARED((NP, LN), jnp.float32),
        pltpu.VMEM((NB, EK), jnp.int32),
        pltpu.VMEM((NB, EK), jnp.int32),
        pltpu.VMEM((NB, EK, D), jnp.float32),
        pltpu.VMEM((RPT, LN), jnp.float32),
        pltpu.VMEM((EK, LN), jnp.float32),
        pltpu.VMEM((RC, D), jnp.float32),
        pltpu.VMEM((RC, D), jnp.float32),
        pltpu.SemaphoreType.DMA,
        pltpu.SemaphoreType.DMA,
        pltpu.SemaphoreType.DMA,
        pltpu.SemaphoreType.DMA,
    ],
    compiler_params=pltpu.CompilerParams(use_tc_tiling_on_sc=False),
)
def _sc_main(xs, srcs, dsts, out, xp, y1p, acc_sh, deg_sh, isrc, idst, rows,
             disb, onesb, arow, brow, sem0, sem1, isem0, isem1):
    _sc_main_body(xs, srcs, dsts, out, xp, y1p, acc_sh, deg_sh, isrc, idst,
                  rows, disb, onesb, arow, brow, sem0, sem1, isem0, isem1)


def _tc_fin_body(oi_ref, o0_ref, o1_ref, f_ref):
    f_ref[...] = oi_ref[...] + o0_ref[0] + o1_ref[0]


def _tc_fin(oi, outp):
    bf = 1000
    return pl.pallas_call(
        _tc_fin_body,
        grid=(N // bf,),
        in_specs=[pl.BlockSpec((bf, D), lambda i: (i, 0)),
                  pl.BlockSpec((1, bf, D), lambda i: (0, i, 0)),
                  pl.BlockSpec((1, bf, D), lambda i: (1, i, 0))],
        out_specs=pl.BlockSpec((bf, D), lambda i: (i, 0)),
        out_shape=jax.ShapeDtypeStruct((N, D), jnp.float32),
    )(oi, outp, outp)


def kernel(user_emb, item_emb, W_u, U_u, ub_u, ib_u, W_r, U_r, ub_r, ib_r,
           W_h, U_h, ub_h, ib_h, edge_index_series):
    f32 = jnp.float32
    x0 = jnp.concatenate(
        [user_emb.astype(f32), item_emb.astype(f32),
         jnp.zeros((NP - N, D), f32)], axis=0)
    bias = jnp.concatenate(
        [ub_u.reshape(1, D), ib_u.reshape(1, D),
         ub_r.reshape(1, D), ib_r.reshape(1, D),
         ub_h.reshape(1, D), ib_h.reshape(1, D),
         jnp.zeros((2, D), f32)], axis=0).astype(f32)

    xs, oi = _tc_prep(x0, W_u.astype(f32), U_u.astype(f32), W_r.astype(f32),
                      U_r.astype(f32), W_h.astype(f32), U_h.astype(f32), bias)

    ei = edge_index_series.astype(jnp.int32)
    pad = jnp.full((T, EP - E), N, jnp.int32)
    srcs = jnp.concatenate([ei[:, 0, :], pad], axis=1)
    dsts = jnp.concatenate([ei[:, 1, :], pad], axis=1)

    outp, _xp, _y1p = _sc_main(xs, srcs, dsts)

    fin = _tc_fin(oi, outp)
    return (fin[:NU], fin[NU:N])
